# baseline (speedup 1.0000x reference)
import functools

import jax
import jax.numpy as jnp
from jax import lax
from jax.experimental import pallas as pl
from jax.experimental.pallas import tpu as pltpu
from jax.experimental.pallas import tpu_sc as plsc

N_NODE = 10000
E_EDGES = 160000
D_IN = 128
HID = 128
HEADS = 8
D_HEAD = 16
OUT = 64
NEG_SLOPE = 0.2

NSUB = 16
CH = 80
NCHUNK = E_EDGES // CH
ROWS_PER_TILE = 640
NPAD = NSUB * ROWS_PER_TILE


def _proj_body(xpi, xai, Wp, bp, Wa, ba, Asw, Adw, Asc, Adc,
               xp_o, xa_o, asw_o, adw_o, asc_o, adc_o):
    xp = jnp.dot(xpi[...], Wp[...], preferred_element_type=jnp.float32) + bp[...]
    xa = jnp.dot(xai[...], Wa[...], preferred_element_type=jnp.float32) + ba[...]
    xp_o[...] = xp
    xa_o[...] = xa
    asw_o[...] = jnp.dot(xa, Asw[...], preferred_element_type=jnp.float32)
    adw_o[...] = jnp.dot(xp, Adw[...], preferred_element_type=jnp.float32)
    asc_o[...] = jnp.dot(xp, Asc[...], preferred_element_type=jnp.float32)
    adc_o[...] = jnp.dot(xp, Adc[...], preferred_element_type=jnp.float32)


def _proj_call(x_paper, x_author, W_paper, b_paper, W_author, b_author,
               Asw, Adw, Asc, Adc):
    f32 = jnp.float32
    outs = [
        jax.ShapeDtypeStruct((N_NODE, HID), f32),
        jax.ShapeDtypeStruct((N_NODE, HID), f32),
        jax.ShapeDtypeStruct((N_NODE, 16), f32),
        jax.ShapeDtypeStruct((N_NODE, 16), f32),
        jax.ShapeDtypeStruct((N_NODE, 16), f32),
        jax.ShapeDtypeStruct((N_NODE, 16), f32),
    ]
    return pl.pallas_call(_proj_body, out_shape=outs)(
        x_paper, x_author, W_paper, b_paper.reshape(1, HID),
        W_author, b_author.reshape(1, HID), Asw, Adw, Asc, Adc)


_GDN = lax.GatherDimensionNumbers(
    offset_dims=(), collapsed_slice_dims=(0,), start_index_map=(0,))


def _lane_splat(vec, h):
    idx = jnp.full((16, 1), h, dtype=jnp.int32)
    return lax.gather(vec, idx, _GDN, (1,),
                      mode=lax.GatherScatterMode.PROMISE_IN_BOUNDS)


def _zeros16():
    return jnp.zeros((16,), jnp.float32)


def _gathers_start(a_src, a_dst, x_src, ei, buf, gsem, c):
    eib, arows, brows, exb, xrows = buf
    base = c * CH
    pltpu.sync_copy(ei.at[:, pl.ds(base, CH)], eib)
    pltpu.async_copy(a_src.at[eib.at[0]], arows, gsem)
    pltpu.async_copy(a_dst.at[eib.at[1]], brows, gsem)
    pltpu.async_copy(x_src.at[eib.at[0]], xrows, gsem)


def _gathers_wait(a_src, a_dst, x_src, buf, gsem):
    eib, arows, brows, exb, xrows = buf
    pltpu.make_async_copy(a_src.at[eib.at[0]], arows, gsem).wait()
    pltpu.make_async_copy(a_dst.at[eib.at[1]], brows, gsem).wait()
    pltpu.make_async_copy(x_src.at[eib.at[0]], xrows, gsem).wait()


def _scatter_start(out_acc, den_acc, buf, ssem):
    eib, arows, brows, exb, xrows = buf
    pltpu.async_copy(exb, den_acc.at[eib.at[1]], ssem, add=True)
    pltpu.async_copy(xrows, out_acc.at[eib.at[1]], ssem, add=True)


def _scatter_wait(out_acc, den_acc, buf, ssem):
    eib, arows, brows, exb, xrows = buf
    pltpu.make_async_copy(exb, den_acc.at[eib.at[1]], ssem).wait()
    pltpu.make_async_copy(xrows, out_acc.at[eib.at[1]], ssem).wait()


def _zero_buf(buf):
    eib, arows, brows, exb, xrows = buf

    @pl.loop(0, CH)
    def _(r):
        exb[r, :] = _zeros16()
        for j in range(HID // 16):
            xrows[r, pl.ds(j * 16, 16)] = _zeros16()


def _compute(buf):
    eib, arows, brows, exb, xrows = buf

    @plsc.parallel_loop(0, CH, unroll=8)
    def _(e):
        al = arows[e, :] + brows[e, :]
        al = jnp.where(al >= 0.0, al, al * NEG_SLOPE)
        ex = jnp.exp(al)
        exb[e, :] = ex
        for h in range(HEADS):
            w = _lane_splat(ex, h)
            xrows[e, pl.ds(h * 16, 16)] = xrows[e, pl.ds(h * 16, 16)] * w


def _run_edge_type(a_src, a_dst, x_src, ei, agg_o, den_o,
                   out_acc, den_acc, bufA, bufB, gsemA, gsemB, ssemA, ssemB,
                   sid):
    nj = NCHUNK // NSUB

    def chunk_of(j):
        return jnp.where(j < nj, sid + j * NSUB, sid)

    _zero_buf(bufA)
    rbase = sid * ROWS_PER_TILE
    for k in range(ROWS_PER_TILE // CH):
        pltpu.sync_copy(bufA[4], out_acc.at[pl.ds(rbase + k * CH, CH)])
        pltpu.sync_copy(bufA[3], den_acc.at[pl.ds(rbase + k * CH, CH)])
    plsc.subcore_barrier()

    _gathers_start(a_src, a_dst, x_src, ei, bufA, gsemA, sid)
    _gathers_start(a_src, a_dst, x_src, ei, bufB, gsemB, sid + NSUB)

    @pl.loop(0, (nj + 1) // 2)
    def _(p):
        jA = 2 * p
        jB = 2 * p + 1
        _gathers_wait(a_src, a_dst, x_src, bufA, gsemA)
        _compute(bufA)
        _scatter_start(out_acc, den_acc, bufA, ssemA)

        _gathers_wait(a_src, a_dst, x_src, bufB, gsemB)
        _compute(bufB)

        @pl.when(jB >= nj)
        def _():
            _zero_buf(bufB)

        _scatter_start(out_acc, den_acc, bufB, ssemB)

        _scatter_wait(out_acc, den_acc, bufA, ssemA)
        _gathers_start(a_src, a_dst, x_src, ei, bufA, gsemA,
                       chunk_of(jA + 2))
        _scatter_wait(out_acc, den_acc, bufB, ssemB)
        _gathers_start(a_src, a_dst, x_src, ei, bufB, gsemB,
                       chunk_of(jB + 2))

    _gathers_wait(a_src, a_dst, x_src, bufA, gsemA)
    _gathers_wait(a_src, a_dst, x_src, bufB, gsemB)
    plsc.subcore_barrier()
    pltpu.sync_copy(out_acc.at[pl.ds(rbase, ROWS_PER_TILE)],
                    agg_o.at[pl.ds(rbase, ROWS_PER_TILE)])
    pltpu.sync_copy(den_acc.at[pl.ds(rbase, ROWS_PER_TILE)],
                    den_o.at[pl.ds(rbase, ROWS_PER_TILE)])


def _sc_edge_call(asw, adw, xa, ei_w, asc, adc, xp, ei_c):
    f32 = jnp.float32
    mesh = plsc.VectorSubcoreMesh(core_axis_name="c", subcore_axis_name="s")

    @functools.partial(
        pl.kernel,
        out_type=[
            jax.ShapeDtypeStruct((NPAD, HID), f32),
            jax.ShapeDtypeStruct((NPAD, 16), f32),
            jax.ShapeDtypeStruct((NPAD, HID), f32),
            jax.ShapeDtypeStruct((NPAD, 16), f32),
        ],
        mesh=mesh,
        compiler_params=pltpu.CompilerParams(use_tc_tiling_on_sc=False),
        scratch_types=[
            pltpu.VMEM_SHARED((NPAD, HID), f32),
            pltpu.VMEM_SHARED((NPAD, 16), f32),
        ] + 2 * [
            pltpu.VMEM((2, CH), jnp.int32),
            pltpu.VMEM((CH, 16), f32),
            pltpu.VMEM((CH, 16), f32),
            pltpu.VMEM((CH, 16), f32),
            pltpu.VMEM((CH, HID), f32),
        ] + 4 * [pltpu.SemaphoreType.DMA],
    )
    def k(asw_h, adw_h, xa_h, eiw_h, asc_h, adc_h, xp_h, eic_h,
          aggw_o, denw_o, aggc_o, denc_o,
          out_acc, den_acc,
          eibA, arowsA, browsA, exbA, xrowsA,
          eibB, arowsB, browsB, exbB, xrowsB,
          gsemA, gsemB, ssemA, ssemB):
        cid = lax.axis_index("c")
        sid = lax.axis_index("s")
        bufA = (eibA, arowsA, browsA, exbA, xrowsA)
        bufB = (eibB, arowsB, browsB, exbB, xrowsB)

        @pl.when(cid == 0)
        def _():
            _run_edge_type(asw_h, adw_h, xa_h, eiw_h, aggw_o, denw_o,
                           out_acc, den_acc, bufA, bufB,
                           gsemA, gsemB, ssemA, ssemB, sid)

        @pl.when(cid == 1)
        def _():
            _run_edge_type(asc_h, adc_h, xp_h, eic_h, aggc_o, denc_o,
                           out_acc, den_acc, bufA, bufB,
                           gsemA, gsemB, ssemA, ssemB, sid)

    return k(asw, adw, xa, ei_w, asc, adc, xp, ei_c)


def _fin_body(aggw, denw, aggc, denc, E16, Wk, bk, q, gw, gb, gms, Wl, bl,
              res_o):
    aw = aggw[: N_NODE, :]
    ac = aggc[: N_NODE, :]
    dw = jnp.dot(denw[: N_NODE, :], E16[...],
                 preferred_element_type=jnp.float32)
    dc = jnp.dot(denc[: N_NODE, :], E16[...],
                 preferred_element_type=jnp.float32)
    ow = jnp.maximum(aw / (dw + 1e-16), 0.0)
    oc = jnp.maximum(ac / (dc + 1e-16), 0.0)

    kw = jnp.tanh(jnp.dot(ow, Wk[...], preferred_element_type=jnp.float32)
                  + bk[...])
    kc = jnp.tanh(jnp.dot(oc, Wk[...], preferred_element_type=jnp.float32)
                  + bk[...])
    sw = jnp.sum(q[...] * jnp.mean(kw, axis=0, keepdims=True))
    sc = jnp.sum(q[...] * jnp.mean(kc, axis=0, keepdims=True))
    m = jnp.maximum(sw, sc)
    ew = jnp.exp(sw - m)
    ec = jnp.exp(sc - m)
    tot = ew + ec
    out = (ew / tot) * ow + (ec / tot) * oc

    mu = jnp.mean(out, axis=0, keepdims=True)
    cent = out - mu * gms[...]
    var = jnp.mean(cent * cent, axis=0, keepdims=True)
    outn = gw[...] * cent * lax.rsqrt(var + 1e-5) + gb[...]
    res_o[...] = jnp.dot(outn, Wl[...],
                         preferred_element_type=jnp.float32) + bl[...]


def _fin_call(aggw, denw, aggc, denc, E16, W_k, b_k, q_sem,
              gn_weight, gn_bias, gn_mean_scale, W_lin, b_lin):
    return pl.pallas_call(
        _fin_body,
        out_shape=jax.ShapeDtypeStruct((N_NODE, OUT), jnp.float32),
    )(aggw, denw, aggc, denc, E16, W_k, b_k.reshape(1, HID),
      q_sem, gn_weight.reshape(1, HID), gn_bias.reshape(1, HID),
      gn_mean_scale.reshape(1, HID), W_lin, b_lin.reshape(1, OUT))


def _blockdiag(att):
    a = att.reshape(HEADS, D_HEAD)
    eye = jnp.eye(HEADS, dtype=a.dtype)
    blk = (a[:, :, None] * eye[:, None, :]).reshape(HID, HEADS)
    return jnp.pad(blk, ((0, 0), (0, 16 - HEADS)))


def kernel(x_paper, x_author, ei_writes, ei_rev, ei_cites, W_paper, b_paper,
           W_author, b_author, att_src_writes, att_dst_writes, att_src_rev,
           att_dst_rev, att_src_cites, att_dst_cites, W_k, b_k, q_sem,
           gn_weight, gn_bias, gn_mean_scale, W_lin, b_lin):
    Asw = _blockdiag(att_src_writes)
    Adw = _blockdiag(att_dst_writes)
    Asc = _blockdiag(att_src_cites)
    Adc = _blockdiag(att_dst_cites)

    xp, xa, asw, adw, asc, adc = _proj_call(
        x_paper, x_author, W_paper, b_paper, W_author, b_author,
        Asw, Adw, Asc, Adc)

    aggw, denw, aggc, denc = _sc_edge_call(
        asw, adw, xa, ei_writes, asc, adc, xp, ei_cites)

    h_of_col = jnp.arange(HID, dtype=jnp.int32) // D_HEAD
    E16 = (jnp.arange(16, dtype=jnp.int32)[:, None] == h_of_col[None, :]
           ).astype(jnp.float32)

    return _fin_call(aggw, denw, aggc, denc, E16, W_k, b_k, q_sem,
                     gn_weight, gn_bias, gn_mean_scale, W_lin, b_lin)


# speedup vs baseline: 1.3498x; 1.3498x over previous
import functools

import jax
import jax.numpy as jnp
from jax import lax
from jax.experimental import pallas as pl
from jax.experimental.pallas import tpu as pltpu
from jax.experimental.pallas import tpu_sc as plsc

N_NODE = 10000
E_EDGES = 160000
D_IN = 128
HID = 128
HEADS = 8
D_HEAD = 16
OUT = 64
NEG_SLOPE = 0.2

NSUB = 16
CH = 80
NCHUNK = E_EDGES // CH
ROWS_PER_TILE = 640
NPAD = NSUB * ROWS_PER_TILE


def _proj_body(xpi, xai, Wp, bp, Wa, ba, Asw, Adw, Asc, Adc,
               xp_o, xa_o, asw_o, adw_o, asc_o, adc_o):
    xp = jnp.dot(xpi[...], Wp[...], preferred_element_type=jnp.float32) + bp[...]
    xa = jnp.dot(xai[...], Wa[...], preferred_element_type=jnp.float32) + ba[...]
    xp_o[...] = xp
    xa_o[...] = xa
    asw_o[...] = jnp.dot(xa, Asw[...], preferred_element_type=jnp.float32)
    adw_o[...] = jnp.dot(xp, Adw[...], preferred_element_type=jnp.float32)
    asc_o[...] = jnp.dot(xp, Asc[...], preferred_element_type=jnp.float32)
    adc_o[...] = jnp.dot(xp, Adc[...], preferred_element_type=jnp.float32)


def _proj_call(x_paper, x_author, W_paper, b_paper, W_author, b_author,
               Asw, Adw, Asc, Adc):
    f32 = jnp.float32
    outs = [
        jax.ShapeDtypeStruct((N_NODE, HID), f32),
        jax.ShapeDtypeStruct((N_NODE, HID), f32),
        jax.ShapeDtypeStruct((N_NODE, 16), f32),
        jax.ShapeDtypeStruct((N_NODE, 16), f32),
        jax.ShapeDtypeStruct((N_NODE, 16), f32),
        jax.ShapeDtypeStruct((N_NODE, 16), f32),
    ]
    return pl.pallas_call(_proj_body, out_shape=outs)(
        x_paper, x_author, W_paper, b_paper.reshape(1, HID),
        W_author, b_author.reshape(1, HID), Asw, Adw, Asc, Adc)


_GDN = lax.GatherDimensionNumbers(
    offset_dims=(), collapsed_slice_dims=(0,), start_index_map=(0,))


def _lane_splat(vec, h):
    idx = jnp.full((16, 1), h, dtype=jnp.int32)
    return lax.gather(vec, idx, _GDN, (1,),
                      mode=lax.GatherScatterMode.PROMISE_IN_BOUNDS)


def _zeros16():
    return jnp.zeros((16,), jnp.float32)


def _gathers_start(a_src, a_dst, x_src, ei, buf, gsem, c):
    eib, arows, brows, exb, xrows = buf
    base = c * CH
    pltpu.sync_copy(ei.at[:, pl.ds(base, CH)], eib)
    pltpu.async_copy(a_src.at[eib.at[0]], arows, gsem)
    pltpu.async_copy(a_dst.at[eib.at[1]], brows, gsem)
    pltpu.async_copy(x_src.at[eib.at[0]], xrows, gsem)


def _gathers_wait(a_src, a_dst, x_src, buf, gsem):
    eib, arows, brows, exb, xrows = buf
    pltpu.make_async_copy(a_src.at[eib.at[0]], arows, gsem).wait()
    pltpu.make_async_copy(a_dst.at[eib.at[1]], brows, gsem).wait()
    pltpu.make_async_copy(x_src.at[eib.at[0]], xrows, gsem).wait()


def _scatter_start(out_acc, den_acc, buf, ssem):
    eib, arows, brows, exb, xrows = buf
    pltpu.async_copy(exb, den_acc.at[eib.at[1]], ssem, add=True)
    pltpu.async_copy(xrows, out_acc.at[eib.at[1]], ssem, add=True)


def _scatter_wait(out_acc, den_acc, buf, ssem):
    eib, arows, brows, exb, xrows = buf
    pltpu.make_async_copy(exb, den_acc.at[eib.at[1]], ssem).wait()
    pltpu.make_async_copy(xrows, out_acc.at[eib.at[1]], ssem).wait()


def _zero_buf(buf):
    eib, arows, brows, exb, xrows = buf

    @pl.loop(0, CH)
    def _(r):
        exb[r, :] = _zeros16()
        for j in range(HID // 16):
            xrows[r, pl.ds(j * 16, 16)] = _zeros16()


def _compute(buf):
    eib, arows, brows, exb, xrows = buf

    @plsc.parallel_loop(0, CH, unroll=4)
    def _(e):
        al = arows[e, :] + brows[e, :]
        al = jnp.where(al >= 0.0, al, al * NEG_SLOPE)
        ex = jnp.exp(al)
        exb[e, :] = ex
        for h in range(HEADS):
            w = _lane_splat(ex, h)
            xrows[e, pl.ds(h * 16, 16)] = xrows[e, pl.ds(h * 16, 16)] * w


def _run_edge_type(a_src, a_dst, x_src, ei, agg_o, den_o,
                   out_acc, den_acc, bufA, bufB, gsemA, gsemB, ssemA, ssemB,
                   sid):
    nj = NCHUNK // NSUB

    def chunk_of(j):
        return jnp.where(j < nj, sid + j * NSUB, sid)

    _zero_buf(bufA)
    rbase = sid * ROWS_PER_TILE
    for k in range(ROWS_PER_TILE // CH):
        pltpu.sync_copy(bufA[4], out_acc.at[pl.ds(rbase + k * CH, CH)])
        pltpu.sync_copy(bufA[3], den_acc.at[pl.ds(rbase + k * CH, CH)])
    plsc.subcore_barrier()

    _gathers_start(a_src, a_dst, x_src, ei, bufA, gsemA, sid)
    _gathers_start(a_src, a_dst, x_src, ei, bufB, gsemB, sid + NSUB)

    @pl.loop(0, (nj + 1) // 2)
    def _(p):
        jA = 2 * p
        jB = 2 * p + 1
        _gathers_wait(a_src, a_dst, x_src, bufA, gsemA)
        _compute(bufA)
        _scatter_start(out_acc, den_acc, bufA, ssemA)

        _gathers_wait(a_src, a_dst, x_src, bufB, gsemB)
        _compute(bufB)

        @pl.when(jB >= nj)
        def _():
            _zero_buf(bufB)

        _scatter_start(out_acc, den_acc, bufB, ssemB)

        _scatter_wait(out_acc, den_acc, bufA, ssemA)
        _gathers_start(a_src, a_dst, x_src, ei, bufA, gsemA,
                       chunk_of(jA + 2))
        _scatter_wait(out_acc, den_acc, bufB, ssemB)
        _gathers_start(a_src, a_dst, x_src, ei, bufB, gsemB,
                       chunk_of(jB + 2))

    _gathers_wait(a_src, a_dst, x_src, bufA, gsemA)
    _gathers_wait(a_src, a_dst, x_src, bufB, gsemB)
    plsc.subcore_barrier()
    pltpu.sync_copy(out_acc.at[pl.ds(rbase, ROWS_PER_TILE)],
                    agg_o.at[pl.ds(rbase, ROWS_PER_TILE)])
    pltpu.sync_copy(den_acc.at[pl.ds(rbase, ROWS_PER_TILE)],
                    den_o.at[pl.ds(rbase, ROWS_PER_TILE)])


def _sc_edge_call(asw, adw, xa, ei_w, asc, adc, xp, ei_c):
    f32 = jnp.float32
    mesh = plsc.VectorSubcoreMesh(core_axis_name="c", subcore_axis_name="s")

    @functools.partial(
        pl.kernel,
        out_type=[
            jax.ShapeDtypeStruct((NPAD, HID), f32),
            jax.ShapeDtypeStruct((NPAD, 16), f32),
            jax.ShapeDtypeStruct((NPAD, HID), f32),
            jax.ShapeDtypeStruct((NPAD, 16), f32),
        ],
        mesh=mesh,
        compiler_params=pltpu.CompilerParams(use_tc_tiling_on_sc=False),
        scratch_types=[
            pltpu.VMEM_SHARED((NPAD, HID), f32),
            pltpu.VMEM_SHARED((NPAD, 16), f32),
        ] + 2 * [
            pltpu.VMEM((2, CH), jnp.int32),
            pltpu.VMEM((CH, 16), f32),
            pltpu.VMEM((CH, 16), f32),
            pltpu.VMEM((CH, 16), f32),
            pltpu.VMEM((CH, HID), f32),
        ] + 4 * [pltpu.SemaphoreType.DMA],
    )
    def k(asw_h, adw_h, xa_h, eiw_h, asc_h, adc_h, xp_h, eic_h,
          aggw_o, denw_o, aggc_o, denc_o,
          out_acc, den_acc,
          eibA, arowsA, browsA, exbA, xrowsA,
          eibB, arowsB, browsB, exbB, xrowsB,
          gsemA, gsemB, ssemA, ssemB):
        cid = lax.axis_index("c")
        sid = lax.axis_index("s")
        bufA = (eibA, arowsA, browsA, exbA, xrowsA)
        bufB = (eibB, arowsB, browsB, exbB, xrowsB)

        @pl.when(cid == 0)
        def _():
            _run_edge_type(asw_h, adw_h, xa_h, eiw_h, aggw_o, denw_o,
                           out_acc, den_acc, bufA, bufB,
                           gsemA, gsemB, ssemA, ssemB, sid)

        @pl.when(cid == 1)
        def _():
            _run_edge_type(asc_h, adc_h, xp_h, eic_h, aggc_o, denc_o,
                           out_acc, den_acc, bufA, bufB,
                           gsemA, gsemB, ssemA, ssemB, sid)

    return k(asw, adw, xa, ei_w, asc, adc, xp, ei_c)


def _fin_body(aggw, denw, aggc, denc, E16, Wk, bk, q, gw, gb, gms, Wl, bl,
              res_o):
    aw = aggw[: N_NODE, :]
    ac = aggc[: N_NODE, :]
    dw = jnp.dot(denw[: N_NODE, :], E16[...],
                 preferred_element_type=jnp.float32)
    dc = jnp.dot(denc[: N_NODE, :], E16[...],
                 preferred_element_type=jnp.float32)
    ow = jnp.maximum(aw / (dw + 1e-16), 0.0)
    oc = jnp.maximum(ac / (dc + 1e-16), 0.0)

    kw = jnp.tanh(jnp.dot(ow, Wk[...], preferred_element_type=jnp.float32)
                  + bk[...])
    kc = jnp.tanh(jnp.dot(oc, Wk[...], preferred_element_type=jnp.float32)
                  + bk[...])
    sw = jnp.sum(q[...] * jnp.mean(kw, axis=0, keepdims=True))
    sc = jnp.sum(q[...] * jnp.mean(kc, axis=0, keepdims=True))
    m = jnp.maximum(sw, sc)
    ew = jnp.exp(sw - m)
    ec = jnp.exp(sc - m)
    tot = ew + ec
    out = (ew / tot) * ow + (ec / tot) * oc

    mu = jnp.mean(out, axis=0, keepdims=True)
    cent = out - mu * gms[...]
    var = jnp.mean(cent * cent, axis=0, keepdims=True)
    outn = gw[...] * cent * lax.rsqrt(var + 1e-5) + gb[...]
    res_o[...] = jnp.dot(outn, Wl[...],
                         preferred_element_type=jnp.float32) + bl[...]


def _fin_call(aggw, denw, aggc, denc, E16, W_k, b_k, q_sem,
              gn_weight, gn_bias, gn_mean_scale, W_lin, b_lin):
    return pl.pallas_call(
        _fin_body,
        out_shape=jax.ShapeDtypeStruct((N_NODE, OUT), jnp.float32),
    )(aggw, denw, aggc, denc, E16, W_k, b_k.reshape(1, HID),
      q_sem, gn_weight.reshape(1, HID), gn_bias.reshape(1, HID),
      gn_mean_scale.reshape(1, HID), W_lin, b_lin.reshape(1, OUT))


def _blockdiag(att):
    a = att.reshape(HEADS, D_HEAD)
    eye = jnp.eye(HEADS, dtype=a.dtype)
    blk = (a[:, :, None] * eye[:, None, :]).reshape(HID, HEADS)
    return jnp.pad(blk, ((0, 0), (0, 16 - HEADS)))


def kernel(x_paper, x_author, ei_writes, ei_rev, ei_cites, W_paper, b_paper,
           W_author, b_author, att_src_writes, att_dst_writes, att_src_rev,
           att_dst_rev, att_src_cites, att_dst_cites, W_k, b_k, q_sem,
           gn_weight, gn_bias, gn_mean_scale, W_lin, b_lin):
    Asw = _blockdiag(att_src_writes)
    Adw = _blockdiag(att_dst_writes)
    Asc = _blockdiag(att_src_cites)
    Adc = _blockdiag(att_dst_cites)

    xp, xa, asw, adw, asc, adc = _proj_call(
        x_paper, x_author, W_paper, b_paper, W_author, b_author,
        Asw, Adw, Asc, Adc)

    aggw, denw, aggc, denc = _sc_edge_call(
        asw, adw, xa, ei_writes, asc, adc, xp, ei_cites)

    h_of_col = jnp.arange(HID, dtype=jnp.int32) // D_HEAD
    E16 = (jnp.arange(16, dtype=jnp.int32)[:, None] == h_of_col[None, :]
           ).astype(jnp.float32)

    return _fin_call(aggw, denw, aggc, denc, E16, W_k, b_k, q_sem,
                     gn_weight, gn_bias, gn_mean_scale, W_lin, b_lin)
